# baseline (speedup 1.0000x reference)
import functools

import jax
import jax.numpy as jnp
from jax import lax
from jax.experimental import pallas as pl
from jax.experimental.pallas import tpu as pltpu
from jax.experimental.pallas import tpu_sc as plsc

N_CORES = 2
N_SUBCORES = 16
LANES = 16
N_WORKERS = N_CORES * N_SUBCORES

BATCH = 16384
FACTORS = 64
B_PER_W = BATCH // N_WORKERS
GROUPS = B_PER_W // LANES

_MESH = plsc.VectorSubcoreMesh(
    core_axis_name="c", subcore_axis_name="s",
    num_cores=N_CORES, num_subcores=N_SUBCORES,
)


def _worker_base():
    wid = lax.axis_index("s") * N_CORES + lax.axis_index("c")
    return pl.multiple_of(wid * B_PER_W, B_PER_W)


@functools.partial(
    pl.kernel,
    out_type=(
        jax.ShapeDtypeStruct((BATCH,), jnp.float32),
        jax.ShapeDtypeStruct((BATCH,), jnp.float32),
    ),
    mesh=_MESH,
    compiler_params=pltpu.CompilerParams(
        needs_layout_passes=False, use_tc_tiling_on_sc=False),
    scratch_types=[
        pltpu.VMEM((B_PER_W,), jnp.int32),
        pltpu.VMEM((B_PER_W,), jnp.int32),
        pltpu.VMEM((B_PER_W,), jnp.float32),
        pltpu.VMEM((B_PER_W,), jnp.float32),
        pltpu.SemaphoreType.DMA,
    ],
)
def _bias_kernel(users_hbm, items_hbm, ubias_hbm, ibias_hbm, ub_out, ib_out,
                 uidx_v, iidx_v, ub_v, ib_v, sem):
    base = _worker_base()
    pltpu.sync_copy(users_hbm.at[pl.ds(base, B_PER_W)], uidx_v)
    pltpu.sync_copy(items_hbm.at[pl.ds(base, B_PER_W)], iidx_v)
    c1 = pltpu.async_copy(ubias_hbm.at[uidx_v], ub_v, sem)
    c2 = pltpu.async_copy(ibias_hbm.at[iidx_v], ib_v, sem)
    c1.wait()
    c2.wait()
    pltpu.sync_copy(ub_v, ub_out.at[pl.ds(base, B_PER_W)])
    pltpu.sync_copy(ib_v, ib_out.at[pl.ds(base, B_PER_W)])


@functools.partial(
    pl.kernel,
    out_type=jax.ShapeDtypeStruct((BATCH,), jnp.float32),
    mesh=_MESH,
    compiler_params=pltpu.CompilerParams(
        needs_layout_passes=False, use_tc_tiling_on_sc=True),
    scratch_types=[
        pltpu.VMEM((B_PER_W,), jnp.int32),
        pltpu.VMEM((B_PER_W,), jnp.int32),
        pltpu.VMEM((B_PER_W,), jnp.int32),
        pltpu.VMEM((B_PER_W,), jnp.int32),
        pltpu.VMEM((B_PER_W // 2, 2 * FACTORS), jnp.float32),
        pltpu.VMEM((B_PER_W // 2, 2 * FACTORS), jnp.float32),
        pltpu.VMEM((B_PER_W,), jnp.float32),
        pltpu.VMEM((B_PER_W,), jnp.float32),
        pltpu.VMEM((B_PER_W,), jnp.float32),
        pltpu.SemaphoreType.DMA,
    ],
)
def _dot_kernel(users_hbm, items_hbm, uemb2_hbm, iemb2_hbm, ubg_hbm, ibg_hbm,
                out_hbm, u_v, i_v, urow_v, irow_v, upair_v, ipair_v,
                ub_v, ib_v, out_v, sem):
    base = _worker_base()
    pltpu.sync_copy(users_hbm.at[pl.ds(base, B_PER_W)], u_v)
    pltpu.sync_copy(items_hbm.at[pl.ds(base, B_PER_W)], i_v)
    pltpu.sync_copy(ubg_hbm.at[pl.ds(base, B_PER_W)], ub_v)
    pltpu.sync_copy(ibg_hbm.at[pl.ds(base, B_PER_W)], ib_v)

    for g in range(GROUPS):
        sl = pl.ds(g * LANES, LANES)
        urow_v[sl] = u_v[sl] >> 1
        irow_v[sl] = i_v[sl] >> 1

    half = B_PER_W // 2
    for h in range(2):
        c1 = pltpu.async_copy(
            uemb2_hbm.at[urow_v.at[pl.ds(h * half, half)]], upair_v, sem)
        c2 = pltpu.async_copy(
            iemb2_hbm.at[irow_v.at[pl.ds(h * half, half)]], ipair_v, sem)
        c1.wait()
        c2.wait()
        for g in range(half // LANES):
            sl = pl.ds(h * half + g * LANES, LANES)
            acc = ub_v[sl] + ib_v[sl]
            rid = g * LANES + lax.iota(jnp.int32, LANES)
            ucol = (u_v[sl] & 1) * FACTORS
            icol = (i_v[sl] & 1) * FACTORS
            for f in range(FACTORS):
                acc = acc + (plsc.load_gather(upair_v, [rid, ucol + f]) *
                             plsc.load_gather(ipair_v, [rid, icol + f]))
            out_v[sl] = acc
    pltpu.sync_copy(out_v, out_hbm.at[pl.ds(base, B_PER_W)])


def kernel(users, items, user_embeddings, item_embeddings, user_biases,
           item_biases):
    users = users.astype(jnp.int32)
    items = items.astype(jnp.int32)
    ubg, ibg = _bias_kernel(
        users, items, user_biases.reshape(-1), item_biases.reshape(-1))
    uemb2 = user_embeddings.reshape(500000, 128)
    iemb2 = item_embeddings.reshape(500000, 128)
    return _dot_kernel(users, items, uemb2, iemb2, ubg, ibg)


# speedup vs baseline: 1.0665x; 1.0665x over previous
import functools

import jax
import jax.numpy as jnp
from jax import lax
from jax.experimental import pallas as pl
from jax.experimental.pallas import tpu as pltpu
from jax.experimental.pallas import tpu_sc as plsc

N_CORES = 2
N_SUBCORES = 16
LANES = 16
N_WORKERS = N_CORES * N_SUBCORES

BATCH = 16384
FACTORS = 64
N_ROWS = 1000000
B_PER_W = BATCH // N_WORKERS
CHUNK = 64
PASSES = B_PER_W // CHUNK
GROUPS_PER_PASS = CHUNK // LANES

_MESH = plsc.VectorSubcoreMesh(
    core_axis_name="c", subcore_axis_name="s",
    num_cores=N_CORES, num_subcores=N_SUBCORES,
)


def _worker_base():
    wid = lax.axis_index("s") * N_CORES + lax.axis_index("c")
    return pl.multiple_of(wid * B_PER_W, B_PER_W)


@functools.partial(
    pl.kernel,
    out_type=(
        jax.ShapeDtypeStruct((BATCH,), jnp.float32),
        jax.ShapeDtypeStruct((BATCH,), jnp.float32),
    ),
    mesh=_MESH,
    compiler_params=pltpu.CompilerParams(
        needs_layout_passes=False, use_tc_tiling_on_sc=False),
    scratch_types=[
        pltpu.VMEM((B_PER_W,), jnp.int32),
        pltpu.VMEM((B_PER_W,), jnp.int32),
        pltpu.VMEM((B_PER_W,), jnp.float32),
        pltpu.VMEM((B_PER_W,), jnp.float32),
        pltpu.SemaphoreType.DMA,
    ],
)
def _bias_kernel(users_hbm, items_hbm, ubias_hbm, ibias_hbm, ub_out, ib_out,
                 uidx_v, iidx_v, ub_v, ib_v, sem):
    base = _worker_base()
    pltpu.sync_copy(users_hbm.at[pl.ds(base, B_PER_W)], uidx_v)
    pltpu.sync_copy(items_hbm.at[pl.ds(base, B_PER_W)], iidx_v)
    c1 = pltpu.async_copy(ubias_hbm.at[uidx_v], ub_v, sem)
    c2 = pltpu.async_copy(ibias_hbm.at[iidx_v], ib_v, sem)
    c1.wait()
    c2.wait()
    pltpu.sync_copy(ub_v, ub_out.at[pl.ds(base, B_PER_W)])
    pltpu.sync_copy(ib_v, ib_out.at[pl.ds(base, B_PER_W)])


@functools.partial(
    pl.kernel,
    out_type=jax.ShapeDtypeStruct((BATCH,), jnp.float32),
    mesh=_MESH,
    compiler_params=pltpu.CompilerParams(
        needs_layout_passes=False, use_tc_tiling_on_sc=True),
    scratch_types=[
        pltpu.VMEM((B_PER_W,), jnp.int32),
        pltpu.VMEM((B_PER_W,), jnp.int32),
        pltpu.VMEM((CHUNK, 128), jnp.float32),
        pltpu.VMEM((CHUNK, 128), jnp.float32),
        pltpu.VMEM((B_PER_W,), jnp.float32),
        pltpu.VMEM((B_PER_W,), jnp.float32),
        pltpu.VMEM((B_PER_W,), jnp.float32),
        pltpu.SemaphoreType.DMA,
    ],
)
def _dot_kernel(users_hbm, items_hbm, upad_hbm, ipad_hbm, ubg_hbm, ibg_hbm,
                out_hbm, u_v, i_v, ubuf_v, ibuf_v, ub_v, ib_v, out_v, sem):
    base = _worker_base()
    pltpu.sync_copy(users_hbm.at[pl.ds(base, B_PER_W)], u_v)
    pltpu.sync_copy(items_hbm.at[pl.ds(base, B_PER_W)], i_v)
    pltpu.sync_copy(ubg_hbm.at[pl.ds(base, B_PER_W)], ub_v)
    pltpu.sync_copy(ibg_hbm.at[pl.ds(base, B_PER_W)], ib_v)

    for h in range(PASSES):
        c1 = pltpu.async_copy(
            upad_hbm.at[u_v.at[pl.ds(h * CHUNK, CHUNK)]], ubuf_v, sem)
        c2 = pltpu.async_copy(
            ipad_hbm.at[i_v.at[pl.ds(h * CHUNK, CHUNK)]], ibuf_v, sem)
        c1.wait()
        c2.wait()

        def group_body(g, _, h=h):
            abs_idx = h * CHUNK + g * LANES + lax.iota(jnp.int32, LANES)
            jloc = g * LANES + lax.iota(jnp.int32, LANES)
            acc = (plsc.load_gather(ub_v, [abs_idx]) +
                   plsc.load_gather(ib_v, [abs_idx]))

            def f_body(f8, acc):
                for k in range(8):
                    fcol = jnp.broadcast_to(f8 * 8 + k, (LANES,))
                    acc = acc + (
                        plsc.load_gather(ubuf_v, [jloc, fcol]) *
                        plsc.load_gather(ibuf_v, [jloc, fcol]))
                return acc

            acc = lax.fori_loop(0, FACTORS // 8, f_body, acc)
            plsc.store_scatter(out_v, [abs_idx], acc)
            return _

        lax.fori_loop(0, GROUPS_PER_PASS, group_body, 0)

    pltpu.sync_copy(out_v, out_hbm.at[pl.ds(base, B_PER_W)])


def kernel(users, items, user_embeddings, item_embeddings, user_biases,
           item_biases):
    users = users.astype(jnp.int32)
    items = items.astype(jnp.int32)
    ubg, ibg = _bias_kernel(
        users, items, user_biases.reshape(-1), item_biases.reshape(-1))
    upad = jnp.pad(user_embeddings, ((0, 0), (0, 128 - FACTORS)))
    ipad = jnp.pad(item_embeddings, ((0, 0), (0, 128 - FACTORS)))
    return _dot_kernel(users, items, upad, ipad, ubg, ibg)


# speedup vs baseline: 1.2055x; 1.1303x over previous
import functools

import jax
import jax.numpy as jnp
from jax import lax
from jax.experimental import pallas as pl
from jax.experimental.pallas import tpu as pltpu
from jax.experimental.pallas import tpu_sc as plsc

N_CORES = 2
N_SUBCORES = 16
LANES = 16
N_WORKERS = N_CORES * N_SUBCORES

BATCH = 16384
FACTORS = 64
B_PER_W = BATCH // N_WORKERS
CHUNK = 64
PASSES = B_PER_W // CHUNK
GROUPS_PER_PASS = CHUNK // LANES

_MESH = plsc.VectorSubcoreMesh(
    core_axis_name="c", subcore_axis_name="s",
    num_cores=N_CORES, num_subcores=N_SUBCORES,
)


def _worker_base():
    wid = lax.axis_index("s") * N_CORES + lax.axis_index("c")
    return pl.multiple_of(wid * B_PER_W, B_PER_W)


@functools.partial(
    pl.kernel,
    out_type=(
        jax.ShapeDtypeStruct((BATCH,), jnp.float32),
        jax.ShapeDtypeStruct((BATCH,), jnp.float32),
    ),
    mesh=_MESH,
    compiler_params=pltpu.CompilerParams(
        needs_layout_passes=False, use_tc_tiling_on_sc=False),
    scratch_types=[
        pltpu.VMEM((B_PER_W,), jnp.int32),
        pltpu.VMEM((B_PER_W,), jnp.int32),
        pltpu.VMEM((B_PER_W,), jnp.float32),
        pltpu.VMEM((B_PER_W,), jnp.float32),
        pltpu.SemaphoreType.DMA,
    ],
)
def _bias_kernel(users_hbm, items_hbm, ubias_hbm, ibias_hbm, ub_out, ib_out,
                 uidx_v, iidx_v, ub_v, ib_v, sem):
    base = _worker_base()
    pltpu.sync_copy(users_hbm.at[pl.ds(base, B_PER_W)], uidx_v)
    pltpu.sync_copy(items_hbm.at[pl.ds(base, B_PER_W)], iidx_v)
    c1 = pltpu.async_copy(ubias_hbm.at[uidx_v], ub_v, sem)
    c2 = pltpu.async_copy(ibias_hbm.at[iidx_v], ib_v, sem)
    c1.wait()
    c2.wait()
    pltpu.sync_copy(ub_v, ub_out.at[pl.ds(base, B_PER_W)])
    pltpu.sync_copy(ib_v, ib_out.at[pl.ds(base, B_PER_W)])


@functools.partial(
    pl.kernel,
    out_type=jax.ShapeDtypeStruct((BATCH,), jnp.float32),
    mesh=_MESH,
    compiler_params=pltpu.CompilerParams(
        needs_layout_passes=False, use_tc_tiling_on_sc=True),
    scratch_types=[
        pltpu.VMEM((B_PER_W,), jnp.int32),
        pltpu.VMEM((B_PER_W,), jnp.int32),
        pltpu.VMEM((CHUNK, 128), jnp.float32),
        pltpu.VMEM((CHUNK, 128), jnp.float32),
        pltpu.VMEM((B_PER_W,), jnp.float32),
        pltpu.VMEM((B_PER_W,), jnp.float32),
        pltpu.VMEM((B_PER_W,), jnp.float32),
        pltpu.SemaphoreType.DMA,
    ],
)
def _dot_kernel(users_hbm, items_hbm, fused_hbm, ubg_hbm, ibg_hbm,
                out_hbm, u_v, i_v, ubuf_v, ibuf_v, ub_v, ib_v, out_v, sem):
    base = _worker_base()
    pltpu.sync_copy(users_hbm.at[pl.ds(base, B_PER_W)], u_v)
    pltpu.sync_copy(items_hbm.at[pl.ds(base, B_PER_W)], i_v)
    pltpu.sync_copy(ubg_hbm.at[pl.ds(base, B_PER_W)], ub_v)
    pltpu.sync_copy(ibg_hbm.at[pl.ds(base, B_PER_W)], ib_v)

    for h in range(PASSES):
        c1 = pltpu.async_copy(
            fused_hbm.at[u_v.at[pl.ds(h * CHUNK, CHUNK)]], ubuf_v, sem)
        c2 = pltpu.async_copy(
            fused_hbm.at[i_v.at[pl.ds(h * CHUNK, CHUNK)]], ibuf_v, sem)
        c1.wait()
        c2.wait()

        def group_body(g, _, h=h):
            abs_idx = h * CHUNK + g * LANES + lax.iota(jnp.int32, LANES)
            jloc = g * LANES + lax.iota(jnp.int32, LANES)
            acc = (plsc.load_gather(ub_v, [abs_idx]) +
                   plsc.load_gather(ib_v, [abs_idx]))

            def f_body(f8, acc):
                for k in range(8):
                    fcol = jnp.broadcast_to(f8 * 8 + k, (LANES,))
                    acc = acc + (
                        plsc.load_gather(ubuf_v, [jloc, fcol]) *
                        plsc.load_gather(ibuf_v, [jloc, fcol + FACTORS]))
                return acc

            acc = lax.fori_loop(0, FACTORS // 8, f_body, acc)
            plsc.store_scatter(out_v, [abs_idx], acc)
            return _

        lax.fori_loop(0, GROUPS_PER_PASS, group_body, 0)

    pltpu.sync_copy(out_v, out_hbm.at[pl.ds(base, B_PER_W)])


def kernel(users, items, user_embeddings, item_embeddings, user_biases,
           item_biases):
    users = users.astype(jnp.int32)
    items = items.astype(jnp.int32)
    ubg, ibg = _bias_kernel(
        users, items, user_biases.reshape(-1), item_biases.reshape(-1))
    fused = jnp.concatenate([user_embeddings, item_embeddings], axis=1)
    return _dot_kernel(users, items, fused, ubg, ibg)
